# baseline (speedup 1.0000x reference)
import jax
import jax.numpy as jnp
from jax.experimental import pallas as pl
from jax.experimental.pallas import tpu as pltpu


_NCHUNK = 64


def _body(w_hbm, o_hbm, buf, in_sems, out_sems):
    n_rows, _ = w_hbm.shape
    batch = o_hbm.shape[0]
    chunk = n_rows // _NCHUNK

    def in_copy(c):
        sl = pl.ds(c * chunk, chunk)
        return pltpu.make_async_copy(w_hbm.at[sl, :], buf.at[sl, :], in_sems.at[c])

    def out_copy(c, b):
        sl = pl.ds(c * chunk, chunk)
        return pltpu.make_async_copy(buf.at[sl, :], o_hbm.at[b, sl, :], out_sems.at[b])

    for c in range(_NCHUNK):
        in_copy(c).start()
    for c in range(_NCHUNK):
        in_copy(c).wait()
        for b in range(batch):
            out_copy(c, b).start()
    for c in range(_NCHUNK):
        for b in range(batch):
            out_copy(c, b).wait()


def kernel(x, pe_weight):
    batch = x.shape[0]
    max_len, d_model = pe_weight.shape
    return pl.pallas_call(
        _body,
        in_specs=[pl.BlockSpec(memory_space=pl.ANY)],
        out_specs=pl.BlockSpec(memory_space=pl.ANY),
        out_shape=jax.ShapeDtypeStruct((batch, max_len, d_model), pe_weight.dtype),
        scratch_shapes=[
            pltpu.VMEM((max_len, d_model), pe_weight.dtype),
            pltpu.SemaphoreType.DMA((_NCHUNK,)),
            pltpu.SemaphoreType.DMA((4,)),
        ],
    )(pe_weight)


# speedup vs baseline: 1.0015x; 1.0015x over previous
import jax
import jax.numpy as jnp
from jax.experimental import pallas as pl
from jax.experimental.pallas import tpu as pltpu


_NCHUNK = 8


def _body(w_hbm, o_hbm, buf, in_sems, out_sems):
    n_rows, _ = w_hbm.shape
    batch = o_hbm.shape[0]
    chunk = n_rows // _NCHUNK

    def in_copy(c):
        sl = pl.ds(c * chunk, chunk)
        return pltpu.make_async_copy(w_hbm.at[sl, :], buf.at[sl, :], in_sems.at[c])

    def out_copy(c, b):
        sl = pl.ds(c * chunk, chunk)
        return pltpu.make_async_copy(buf.at[sl, :], o_hbm.at[b, sl, :], out_sems.at[b])

    for c in range(_NCHUNK):
        in_copy(c).start()
    for c in range(_NCHUNK):
        in_copy(c).wait()
        for b in range(batch):
            out_copy(c, b).start()
    for c in range(_NCHUNK):
        for b in range(batch):
            out_copy(c, b).wait()


def kernel(x, pe_weight):
    batch = x.shape[0]
    max_len, d_model = pe_weight.shape
    return pl.pallas_call(
        _body,
        in_specs=[pl.BlockSpec(memory_space=pl.ANY)],
        out_specs=pl.BlockSpec(memory_space=pl.ANY),
        out_shape=jax.ShapeDtypeStruct((batch, max_len, d_model), pe_weight.dtype),
        scratch_shapes=[
            pltpu.VMEM((max_len, d_model), pe_weight.dtype),
            pltpu.SemaphoreType.DMA((_NCHUNK,)),
            pltpu.SemaphoreType.DMA((4,)),
        ],
    )(pe_weight)
